# Initial kernel scaffold; baseline (speedup 1.0000x reference)
import jax
import jax.numpy as jnp
from jax.experimental import pallas as pl


def kernel(features_d, features_m, W1_d, W2_d, W1_m, W2_m, W_cls0, b_cls0, W_cls, b_cls, W_mf, edge_index_d, edge_index_m, pos_cla_idx, tr_pos, tr_neg):
    raise NotImplementedError("write your pallas kernel here")


# speedup vs baseline: 9.0121x; 9.0121x over previous
import functools

import jax
import jax.numpy as jnp
from jax import lax
from jax.experimental import pallas as pl
from jax.experimental.pallas import tpu as pltpu
from jax.experimental.pallas import tpu_sc as plsc

_N = 10000
_NP = 10240
_E = 320000
_NC = 2
_NS = 16
_NW = _NC * _NS
_EW = _E // _NW
_CH = 80
_NCHUNK = _EW // _CH
_RPT = _NP // _NS
_ZR = 128

_G = 4096
_B_GATHER = 512 + 2 * _G
_BW = _B_GATHER // _NW


def _sc_mesh():
    return plsc.VectorSubcoreMesh(core_axis_name="c", subcore_axis_name="s")


def _make_hist():
    @functools.partial(
        pl.kernel,
        mesh=_sc_mesh(),
        compiler_params=pltpu.CompilerParams(use_tc_tiling_on_sc=False),
        out_type=jax.ShapeDtypeStruct((_NC * 4 * _NP,), jnp.float32),
        scratch_types=[
            pltpu.VMEM((_NCHUNK, _CH), jnp.int32),
            pltpu.VMEM((_CH,), jnp.float32),
            pltpu.VMEM((_RPT,), jnp.float32),
            pltpu.VMEM_SHARED((_NP,), jnp.float32),
            pltpu.VMEM_SHARED((_NP,), jnp.float32),
            pltpu.VMEM_SHARED((_NP,), jnp.float32),
            pltpu.VMEM_SHARED((_NP,), jnp.float32),
        ],
    )
    def hist_kernel(idx_hbm, out_hbm, idxv, onesv, zv, h0, h1, h2, h3):
        hists = (h0, h1, h2, h3)
        cid = lax.axis_index("c")
        sid = lax.axis_index("s")
        wid = cid * _NS + sid

        one16 = jnp.ones((16,), jnp.float32)
        zero16 = jnp.zeros((16,), jnp.float32)
        for k in range(_CH // 16):
            onesv[pl.ds(k * 16, 16)] = one16

        def zrow(i, carry):
            zv[pl.ds(i * 16, 16)] = zero16
            return carry
        lax.fori_loop(0, _RPT // 16, zrow, 0)
        for a in range(4):
            pltpu.sync_copy(zv, hists[a].at[pl.ds(sid * _RPT, _RPT)])
        plsc.subcore_barrier()

        for a in range(4):
            pltpu.sync_copy(idx_hbm.at[a, wid], idxv)

            def body(c, carry):
                pltpu.sync_copy(onesv, hists[a].at[idxv.at[c]], add=True)
                return carry
            lax.fori_loop(0, _NCHUNK, body, 0)

        plsc.subcore_barrier()
        for a in range(4):
            off = (cid * 4 + a) * _NP + sid * _RPT
            pltpu.sync_copy(hists[a].at[pl.ds(sid * _RPT, _RPT)],
                            out_hbm.at[pl.ds(off, _RPT)])

    return hist_kernel


def _make_prop(d_feat):
    @functools.partial(
        pl.kernel,
        mesh=_sc_mesh(),
        compiler_params=pltpu.CompilerParams(use_tc_tiling_on_sc=False),
        out_type=jax.ShapeDtypeStruct((_NC * _NP, d_feat), jnp.float32),
        scratch_types=[
            pltpu.VMEM((_NCHUNK, _CH), jnp.int32),
            pltpu.VMEM((_NCHUNK, _CH), jnp.int32),
            pltpu.VMEM((_CH, d_feat), jnp.float32),
            pltpu.VMEM((_ZR, d_feat), jnp.float32),
            pltpu.VMEM_SHARED((_NP, d_feat), jnp.float32),
            pltpu.SemaphoreType.DMA,
        ],
    )
    def prop_kernel(ys_hbm, src_hbm, dst_hbm, out_hbm,
                    sidx, didx, rows, zrows, acc, sem0):
        cid = lax.axis_index("c")
        sid = lax.axis_index("s")

        zero16 = jnp.zeros((16,), jnp.float32)

        def zrow(i, carry):
            for k in range(d_feat // 16):
                zrows[i, pl.ds(k * 16, 16)] = zero16
            return carry
        lax.fori_loop(0, _ZR, zrow, 0)
        for r in range(_RPT // _ZR):
            pltpu.sync_copy(
                zrows, acc.at[pl.ds(sid * _RPT + r * _ZR, _ZR)])
        plsc.subcore_barrier()

        wid = cid * _NS + sid
        pltpu.sync_copy(src_hbm.at[wid], sidx)
        pltpu.sync_copy(dst_hbm.at[wid], didx)

        def body(c, carry):
            pltpu.async_copy(ys_hbm.at[sidx.at[c]], rows, sem0).wait()
            pltpu.sync_copy(rows, acc.at[didx.at[c]], add=True)
            return carry

        lax.fori_loop(0, _NCHUNK, body, 0)

        plsc.subcore_barrier()
        pltpu.sync_copy(acc.at[pl.ds(sid * _RPT, _RPT)],
                        out_hbm.at[pl.ds(cid * _NP + sid * _RPT, _RPT)])

    return prop_kernel


def _make_gather():
    @functools.partial(
        pl.kernel,
        mesh=_sc_mesh(),
        compiler_params=pltpu.CompilerParams(use_tc_tiling_on_sc=False),
        out_type=jax.ShapeDtypeStruct((_B_GATHER, 64), jnp.float32),
        scratch_types=[
            pltpu.VMEM((_BW,), jnp.int32),
            pltpu.VMEM((_BW, 64), jnp.float32),
            pltpu.SemaphoreType.DMA,
        ],
    )
    def gather_kernel(table_hbm, idx_hbm, out_hbm, idxv, rowsv, sem):
        cid = lax.axis_index("c")
        sid = lax.axis_index("s")
        wid = cid * _NS + sid
        base = wid * _BW
        pltpu.sync_copy(idx_hbm.at[pl.ds(base, _BW)], idxv)
        for c in range(0, _BW, _CH):
            n = min(_CH, _BW - c)
            pltpu.async_copy(table_hbm.at[idxv.at[pl.ds(c, n)]],
                             rowsv.at[pl.ds(c, n)], sem).wait()
        pltpu.sync_copy(rowsv, out_hbm.at[pl.ds(base, _BW)])

    return gather_kernel


def _norms_tc(histp):
    def body(h_ref, o_ref):
        deg = h_ref[0] + h_ref[1]
        o_ref[...] = lax.rsqrt(deg + 1.0)
    return pl.pallas_call(
        body,
        out_shape=jax.ShapeDtypeStruct((4, _NP), jnp.float32),
    )(histp)


def _layer1_tc(x, w, rs):
    blk = 1024

    def body(x_ref, w_ref, rs_ref, o_ref):
        t = jnp.dot(x_ref[0], w_ref[0], preferred_element_type=jnp.float32)
        o_ref[0] = t * rs_ref[0]

    return pl.pallas_call(
        body,
        grid=(2, _NP // blk),
        in_specs=[
            pl.BlockSpec((1, blk, 128), lambda g, i: (g, i, 0)),
            pl.BlockSpec((1, 128, 128), lambda g, i: (g, 0, 0)),
            pl.BlockSpec((1, blk, 1), lambda g, i: (g, i, 0)),
        ],
        out_specs=pl.BlockSpec((1, blk, 128), lambda g, i: (g, i, 0)),
        out_shape=jax.ShapeDtypeStruct((2, _NP, 128), jnp.float32),
    )(x, w, rs)


def _layer2_tc(qa, qb, w2a, w2b, rd, rs):
    blk = 1024

    def body(qa_ref, qb_ref, wa_ref, wb_ref, rd_ref, rs_ref, o_ref):
        ha = jax.nn.relu(rd_ref[0] * (qa_ref[0, 0] + qa_ref[0, 1]))
        hb = jax.nn.relu(rd_ref[0] * (qb_ref[0, 0] + qb_ref[0, 1]))
        t = (jnp.dot(ha, wa_ref[0], preferred_element_type=jnp.float32)
             + jnp.dot(hb, wb_ref[0], preferred_element_type=jnp.float32))
        o_ref[0] = t * rs_ref[0]

    return pl.pallas_call(
        body,
        grid=(2, _NP // blk),
        in_specs=[
            pl.BlockSpec((1, _NC, blk, 64), lambda g, i: (g, 0, i, 0)),
            pl.BlockSpec((1, _NC, blk, 64), lambda g, i: (g, 0, i, 0)),
            pl.BlockSpec((1, 64, 64), lambda g, i: (g, 0, 0)),
            pl.BlockSpec((1, 64, 64), lambda g, i: (g, 0, 0)),
            pl.BlockSpec((1, blk, 1), lambda g, i: (g, i, 0)),
            pl.BlockSpec((1, blk, 1), lambda g, i: (g, i, 0)),
        ],
        out_specs=pl.BlockSpec((1, blk, 64), lambda g, i: (g, i, 0)),
        out_shape=jax.ShapeDtypeStruct((2, _NP, 64), jnp.float32),
    )(qa, qb, w2a, w2b, rd, rs)


def _emb_tc(q, rd, wmf2):
    blk = 1024

    def body(q_ref, rd_ref, w_ref, e_ref, p_ref):
        e = rd_ref[0] * (q_ref[0, 0] + q_ref[0, 1])
        e_ref[0] = e
        p_ref[0] = jnp.dot(e, w_ref[0], preferred_element_type=jnp.float32)

    return pl.pallas_call(
        body,
        grid=(2, _NP // blk),
        in_specs=[
            pl.BlockSpec((1, _NC, blk, 64), lambda g, i: (g, 0, i, 0)),
            pl.BlockSpec((1, blk, 1), lambda g, i: (g, i, 0)),
            pl.BlockSpec((1, 64, 64), lambda g, i: (g, 0, 0)),
        ],
        out_specs=[
            pl.BlockSpec((1, blk, 64), lambda g, i: (g, i, 0)),
            pl.BlockSpec((1, blk, 64), lambda g, i: (g, i, 0)),
        ],
        out_shape=[
            jax.ShapeDtypeStruct((2, _NP, 64), jnp.float32),
            jax.ShapeDtypeStruct((2, _NP, 64), jnp.float32),
        ],
    )(q, rd, wmf2)


def _recon_tc(emb):
    bm = 400

    def body(a_ref, b_ref, o_ref):
        o_ref[...] = lax.dot_general(
            a_ref[...], b_ref[...], (((1,), (1,)), ((), ())),
            preferred_element_type=jnp.float32)

    return pl.pallas_call(
        body,
        grid=(_N // bm,),
        in_specs=[
            pl.BlockSpec((bm, 64), lambda i: (i, 0)),
            pl.BlockSpec((_N, 64), lambda i: (0, 0)),
        ],
        out_specs=pl.BlockSpec((bm, _N), lambda i: (i, 0)),
        out_shape=jax.ShapeDtypeStruct((_N, _N), jnp.float32),
    )(emb, emb)


def _head_tc(fea, w0, b0, w1, b1):
    def body(f_ref, w0_ref, b0_ref, w1_ref, b1_ref, o_ref):
        c1 = jax.nn.relu(
            jnp.dot(f_ref[...], w0_ref[...],
                    preferred_element_type=jnp.float32) + b0_ref[0])
        o_ref[...] = jnp.dot(
            c1, w1_ref[...], preferred_element_type=jnp.float32) + b1_ref[0]

    return pl.pallas_call(
        body,
        out_shape=jax.ShapeDtypeStruct((512, 14), jnp.float32),
    )(fea, w0, b0.reshape(1, -1), w1, b1.reshape(1, -1))


def _pairdot_tc(pr, mr):
    def body(a_ref, b_ref, o_ref):
        o_ref[...] = jnp.sum(a_ref[...] * b_ref[...], axis=1, keepdims=True)

    return pl.pallas_call(
        body,
        out_shape=jax.ShapeDtypeStruct((_G, 1), jnp.float32),
    )(pr, mr)


def kernel(features_d, features_m, W1_d, W2_d, W1_m, W2_m, W_cls0, b_cls0,
           W_cls, b_cls, W_mf, edge_index_d, edge_index_m, pos_cla_idx,
           tr_pos, tr_neg):
    src_d = edge_index_d[0].astype(jnp.int32).reshape(_NW, _NCHUNK, _CH)
    dst_d = edge_index_d[1].astype(jnp.int32).reshape(_NW, _NCHUNK, _CH)
    src_m = edge_index_m[0].astype(jnp.int32).reshape(_NW, _NCHUNK, _CH)
    dst_m = edge_index_m[1].astype(jnp.int32).reshape(_NW, _NCHUNK, _CH)
    idx4 = jnp.stack([src_d, dst_d, src_m, dst_m])

    histp = _make_hist()(idx4).reshape(_NC, 4, _NP)
    norms = _norms_tc(histp)
    rs = jnp.stack([norms[0], norms[2]])[..., None]
    rd = jnp.stack([norms[1], norms[3]])[..., None]

    pad = ((0, 0), (0, _NP - _N), (0, 0))
    x = jnp.pad(jnp.stack([features_d, features_m]), pad)
    w1 = jnp.stack([W1_d, W1_m])
    w2 = jnp.stack([W2_d, W2_m])

    ys1 = _layer1_tc(x, w1, rs)

    prop64 = _make_prop(64)

    def prop(ys, s, t):
        return prop64(ys, s, t).reshape(_NC, _NP, 64)

    ys1a = ys1[:, :, :64]
    ys1b = ys1[:, :, 64:]
    q1a = jnp.stack([prop(ys1a[0], src_d, dst_d), prop(ys1a[1], src_m, dst_m)])
    q1b = jnp.stack([prop(ys1b[0], src_d, dst_d), prop(ys1b[1], src_m, dst_m)])

    w2a = w2[:, :64]
    w2b = w2[:, 64:]
    ys2 = _layer2_tc(q1a, q1b, w2a, w2b, rd, rs)

    q2 = jnp.stack([prop(ys2[0], src_d, dst_d), prop(ys2[1], src_m, dst_m)])

    wmf2 = jnp.stack([W_mf, W_mf])
    emb, pd = _emb_tc(q2, rd, wmf2)

    recon_d = _recon_tc(emb[0, :_N])
    recon_m = _recon_tc(emb[1, :_N])

    table = jnp.concatenate([emb[0, :_N], pd[0, :_N], emb[1, :_N]], axis=0)
    pi = jnp.concatenate([tr_pos[:, 0], tr_neg[:, 0]]).astype(jnp.int32)
    mj = jnp.concatenate([tr_pos[:, 1], tr_neg[:, 1]]).astype(jnp.int32)
    gidx = jnp.concatenate([pos_cla_idx.astype(jnp.int32),
                            pi + _N, mj + 2 * _N])
    rows = _make_gather()(table, gidx)

    pred_class = _head_tc(rows[:512], W_cls0, b_cls0, W_cls, b_cls)
    preds_dm = _pairdot_tc(rows[512:512 + _G], rows[512 + _G:])[:, 0]

    return (pred_class, recon_d, recon_m, preds_dm)


# speedup vs baseline: 16.2180x; 1.7996x over previous
import functools

import jax
import jax.numpy as jnp
from jax import lax
from jax.experimental import pallas as pl
from jax.experimental.pallas import tpu as pltpu
from jax.experimental.pallas import tpu_sc as plsc

_N = 10000
_NP = 10240
_E = 320000
_NC = 2
_NS = 16
_NW = _NC * _NS
_EW = _E // _NW
_CH = 80
_NCHUNK = _EW // _CH
_RPT = _NP // _NS
_ZR = 128
_NBUF = 5

_G = 4096
_B_GATHER = 512 + 2 * _G
_BW = _B_GATHER // _NW


def _sc_mesh():
    return plsc.VectorSubcoreMesh(core_axis_name="c", subcore_axis_name="s")

_SC_PARAMS = dict(
    mesh=_sc_mesh(),
    compiler_params=pltpu.CompilerParams(use_tc_tiling_on_sc=False),
)


def _make_hist():
    @functools.partial(
        pl.kernel,
        out_type=jax.ShapeDtypeStruct((_NC * 4 * _NP,), jnp.float32),
        scratch_types=[
            pltpu.VMEM((_NCHUNK, _CH), jnp.int32),
            pltpu.VMEM((_NCHUNK, _CH), jnp.int32),
            pltpu.VMEM((_NCHUNK, _CH), jnp.int32),
            pltpu.VMEM((_NCHUNK, _CH), jnp.int32),
            pltpu.VMEM((_CH,), jnp.float32),
            pltpu.VMEM((_RPT,), jnp.float32),
            pltpu.VMEM_SHARED((_NP,), jnp.float32),
            pltpu.VMEM_SHARED((_NP,), jnp.float32),
            pltpu.VMEM_SHARED((_NP,), jnp.float32),
            pltpu.VMEM_SHARED((_NP,), jnp.float32),
            pltpu.SemaphoreType.DMA,
        ],
        **_SC_PARAMS,
    )
    def hist_kernel(idx_hbm, out_hbm, i0, i1, i2, i3, onesv, zv,
                    h0, h1, h2, h3, sem):
        idxs = (i0, i1, i2, i3)
        hists = (h0, h1, h2, h3)
        cid = lax.axis_index("c")
        sid = lax.axis_index("s")
        wid = cid * _NS + sid

        one16 = jnp.ones((16,), jnp.float32)
        zero16 = jnp.zeros((16,), jnp.float32)
        for k in range(_CH // 16):
            onesv[pl.ds(k * 16, 16)] = one16

        def zrow(i, carry):
            zv[pl.ds(i * 16, 16)] = zero16
            return carry
        lax.fori_loop(0, _RPT // 16, zrow, 0)
        for a in range(4):
            pltpu.sync_copy(zv, hists[a].at[pl.ds(sid * _RPT, _RPT)])
            pltpu.sync_copy(idx_hbm.at[a, wid], idxs[a])
        plsc.subcore_barrier()

        for a in range(4):
            def fire(c, carry):
                pltpu.async_copy(onesv, hists[a].at[idxs[a].at[c]], sem,
                                 add=True)
                return carry
            lax.fori_loop(0, _NCHUNK, fire, 0)

        def drain(c, carry):
            pltpu.make_async_copy(onesv, h0.at[i0.at[0]], sem).wait()
            return carry
        lax.fori_loop(0, 4 * _NCHUNK, drain, 0)

        plsc.subcore_barrier()
        for a in range(4):
            off = (cid * 4 + a) * _NP + sid * _RPT
            pltpu.sync_copy(hists[a].at[pl.ds(sid * _RPT, _RPT)],
                            out_hbm.at[pl.ds(off, _RPT)])

    return hist_kernel


def _make_prop(npass, graph_of_pass):
    @functools.partial(
        pl.kernel,
        out_type=jax.ShapeDtypeStruct((npass * _NC * _NP, 64), jnp.float32),
        scratch_types=[
            pltpu.VMEM((_NCHUNK, _CH), jnp.int32),
            pltpu.VMEM((_NCHUNK, _CH), jnp.int32),
            pltpu.VMEM((_NBUF, _CH, 64), jnp.float32),
            pltpu.VMEM((_ZR, 64), jnp.float32),
            pltpu.VMEM_SHARED((_NP, 64), jnp.float32),
            [pltpu.SemaphoreType.DMA] * _NBUF,
        ],
        **_SC_PARAMS,
    )
    def prop_kernel(ya_hbm, yb_hbm, idx_hbm, out_hbm,
                    sidx, didx, rows, zrows, acc, sems):
        cid = lax.axis_index("c")
        sid = lax.axis_index("s")
        wid = cid * _NS + sid

        zero16 = jnp.zeros((16,), jnp.float32)

        def zrow(i, carry):
            for k in range(64 // 16):
                zrows[i, pl.ds(k * 16, 16)] = zero16
            return carry
        lax.fori_loop(0, _ZR, zrow, 0)
        for r in range(_RPT // _ZR):
            pltpu.sync_copy(zrows, acc.at[pl.ds(sid * _RPT + r * _ZR, _ZR)])
        plsc.subcore_barrier()

        prev_graph = None
        for p in range(npass):
            g = graph_of_pass[p]
            tbl = (ya_hbm if (npass == 2 or p % 2 == 0) else yb_hbm).at[g]
            if g != prev_graph:
                pltpu.sync_copy(idx_hbm.at[2 * g, wid], sidx)
                pltpu.sync_copy(idx_hbm.at[2 * g + 1, wid], didx)
                prev_graph = g

            for b in range(_NBUF - 1):
                pltpu.async_copy(tbl.at[sidx.at[b]], rows.at[b], sems[b])

            def group(gi, carry):
                c0 = gi * _NBUF
                for j in range(_NBUF):
                    c = c0 + j
                    pltpu.make_async_copy(tbl.at[sidx.at[c]], rows.at[j],
                                          sems[j]).wait()
                    pltpu.sync_copy(rows.at[j], acc.at[didx.at[c]], add=True)
                    nb = (j + _NBUF - 1) % _NBUF

                    @pl.when(c + _NBUF - 1 < _NCHUNK)
                    def _():
                        pltpu.async_copy(tbl.at[sidx.at[c + _NBUF - 1]],
                                         rows.at[nb], sems[nb])
                return carry

            lax.fori_loop(0, _NCHUNK // _NBUF, group, 0)

            plsc.subcore_barrier()
            base = (p * _NC + cid) * _NP + sid * _RPT
            pltpu.sync_copy(acc.at[pl.ds(sid * _RPT, _RPT)],
                            out_hbm.at[pl.ds(base, _RPT)])
            if p + 1 < npass:
                for r in range(_RPT // _ZR):
                    pltpu.sync_copy(
                        zrows, acc.at[pl.ds(sid * _RPT + r * _ZR, _ZR)])
                plsc.subcore_barrier()

    return prop_kernel


def _make_gather():
    @functools.partial(
        pl.kernel,
        out_type=jax.ShapeDtypeStruct((_B_GATHER, 64), jnp.float32),
        scratch_types=[
            pltpu.VMEM((_BW,), jnp.int32),
            pltpu.VMEM((_BW, 64), jnp.float32),
            pltpu.SemaphoreType.DMA,
        ],
        **_SC_PARAMS,
    )
    def gather_kernel(table_hbm, idx_hbm, out_hbm, idxv, rowsv, sem):
        cid = lax.axis_index("c")
        sid = lax.axis_index("s")
        wid = cid * _NS + sid
        base = wid * _BW
        pltpu.sync_copy(idx_hbm.at[pl.ds(base, _BW)], idxv)
        chunks = []
        for c in range(0, _BW, _CH):
            n = min(_CH, _BW - c)
            chunks.append(
                pltpu.async_copy(table_hbm.at[idxv.at[pl.ds(c, n)]],
                                 rowsv.at[pl.ds(c, n)], sem))
        for h in chunks:
            h.wait()
        pltpu.sync_copy(rowsv, out_hbm.at[pl.ds(base, _BW)])

    return gather_kernel


def _norms_tc(histp):
    def body(h_ref, o_ref):
        deg = h_ref[0] + h_ref[1]
        o_ref[...] = lax.rsqrt(deg + 1.0)
    return pl.pallas_call(
        body,
        out_shape=jax.ShapeDtypeStruct((4, _NP), jnp.float32),
    )(histp)


def _layer1_tc(x, w, rs):
    blk = 1024

    def body(x_ref, w_ref, rs_ref, oa_ref, ob_ref):
        t = jnp.dot(x_ref[0], w_ref[0],
                    preferred_element_type=jnp.float32) * rs_ref[0]
        oa_ref[0] = t[:, :64]
        ob_ref[0] = t[:, 64:]

    return pl.pallas_call(
        body,
        grid=(2, _NP // blk),
        in_specs=[
            pl.BlockSpec((1, blk, 128), lambda g, i: (g, i, 0)),
            pl.BlockSpec((1, 128, 128), lambda g, i: (g, 0, 0)),
            pl.BlockSpec((1, blk, 1), lambda g, i: (g, i, 0)),
        ],
        out_specs=[
            pl.BlockSpec((1, blk, 64), lambda g, i: (g, i, 0)),
            pl.BlockSpec((1, blk, 64), lambda g, i: (g, i, 0)),
        ],
        out_shape=[
            jax.ShapeDtypeStruct((2, _NP, 64), jnp.float32),
            jax.ShapeDtypeStruct((2, _NP, 64), jnp.float32),
        ],
    )(x, w, rs)


def _layer2_tc(q14, w2a, w2b, rd, rs):
    blk = 1024

    def body(qa_ref, qb_ref, wa_ref, wb_ref, rd_ref, rs_ref, o_ref):
        ha = jax.nn.relu(rd_ref[0] * (qa_ref[0, 0] + qa_ref[0, 1]))
        hb = jax.nn.relu(rd_ref[0] * (qb_ref[0, 0] + qb_ref[0, 1]))
        t = (jnp.dot(ha, wa_ref[0], preferred_element_type=jnp.float32)
             + jnp.dot(hb, wb_ref[0], preferred_element_type=jnp.float32))
        o_ref[0] = t * rs_ref[0]

    return pl.pallas_call(
        body,
        grid=(2, _NP // blk),
        in_specs=[
            pl.BlockSpec((1, _NC, blk, 64), lambda g, i: (2 * g, 0, i, 0)),
            pl.BlockSpec((1, _NC, blk, 64), lambda g, i: (2 * g + 1, 0, i, 0)),
            pl.BlockSpec((1, 64, 64), lambda g, i: (g, 0, 0)),
            pl.BlockSpec((1, 64, 64), lambda g, i: (g, 0, 0)),
            pl.BlockSpec((1, blk, 1), lambda g, i: (g, i, 0)),
            pl.BlockSpec((1, blk, 1), lambda g, i: (g, i, 0)),
        ],
        out_specs=pl.BlockSpec((1, blk, 64), lambda g, i: (g, i, 0)),
        out_shape=jax.ShapeDtypeStruct((2, _NP, 64), jnp.float32),
    )(q14, q14, w2a, w2b, rd, rs)


def _emb_tc(q2, rd, wmf2):
    blk = 1024

    def body(q_ref, rd_ref, w_ref, e_ref, p_ref):
        e = rd_ref[0] * (q_ref[0, 0] + q_ref[0, 1])
        e_ref[0] = e
        p_ref[0] = jnp.dot(e, w_ref[0], preferred_element_type=jnp.float32)

    return pl.pallas_call(
        body,
        grid=(2, _NP // blk),
        in_specs=[
            pl.BlockSpec((1, _NC, blk, 64), lambda g, i: (g, 0, i, 0)),
            pl.BlockSpec((1, blk, 1), lambda g, i: (g, i, 0)),
            pl.BlockSpec((1, 64, 64), lambda g, i: (g, 0, 0)),
        ],
        out_specs=[
            pl.BlockSpec((1, blk, 64), lambda g, i: (g, i, 0)),
            pl.BlockSpec((1, blk, 64), lambda g, i: (g, i, 0)),
        ],
        out_shape=[
            jax.ShapeDtypeStruct((2, _NP, 64), jnp.float32),
            jax.ShapeDtypeStruct((2, _NP, 64), jnp.float32),
        ],
    )(q2, rd, wmf2)


def _recon_tc(emb):
    bm = 400

    def body(a_ref, b_ref, o_ref):
        o_ref[...] = lax.dot_general(
            a_ref[...], b_ref[...], (((1,), (1,)), ((), ())),
            preferred_element_type=jnp.float32)

    return pl.pallas_call(
        body,
        grid=(_N // bm,),
        in_specs=[
            pl.BlockSpec((bm, 64), lambda i: (i, 0)),
            pl.BlockSpec((_N, 64), lambda i: (0, 0)),
        ],
        out_specs=pl.BlockSpec((bm, _N), lambda i: (i, 0)),
        out_shape=jax.ShapeDtypeStruct((_N, _N), jnp.float32),
    )(emb, emb)


def _head_tc(fea, w0, b0, w1, b1):
    def body(f_ref, w0_ref, b0_ref, w1_ref, b1_ref, o_ref):
        c1 = jax.nn.relu(
            jnp.dot(f_ref[...], w0_ref[...],
                    preferred_element_type=jnp.float32) + b0_ref[0])
        o_ref[...] = jnp.dot(
            c1, w1_ref[...], preferred_element_type=jnp.float32) + b1_ref[0]

    return pl.pallas_call(
        body,
        out_shape=jax.ShapeDtypeStruct((512, 14), jnp.float32),
    )(fea, w0, b0.reshape(1, -1), w1, b1.reshape(1, -1))


def _pairdot_tc(pr, mr):
    def body(a_ref, b_ref, o_ref):
        o_ref[...] = jnp.sum(a_ref[...] * b_ref[...], axis=1, keepdims=True)

    return pl.pallas_call(
        body,
        out_shape=jax.ShapeDtypeStruct((_G, 1), jnp.float32),
    )(pr, mr)


def kernel(features_d, features_m, W1_d, W2_d, W1_m, W2_m, W_cls0, b_cls0,
           W_cls, b_cls, W_mf, edge_index_d, edge_index_m, pos_cla_idx,
           tr_pos, tr_neg):
    ei = jnp.stack([edge_index_d, edge_index_m]).astype(jnp.int32)
    idx4 = ei.reshape(4, _NW, _NCHUNK, _CH)

    histp = _make_hist()(idx4).reshape(_NC, 4, _NP)
    norms = _norms_tc(histp)
    rs = jnp.stack([norms[0], norms[2]])[..., None]
    rd = jnp.stack([norms[1], norms[3]])[..., None]

    pad = ((0, 0), (0, _NP - _N), (0, 0))
    x = jnp.pad(jnp.stack([features_d, features_m]), pad)
    w1 = jnp.stack([W1_d, W1_m])
    w2 = jnp.stack([W2_d, W2_m])

    ya, yb = _layer1_tc(x, w1, rs)

    q14 = _make_prop(4, (0, 0, 1, 1))(ya, yb, idx4)
    q14 = q14.reshape(4, _NC, _NP, 64)

    ys2 = _layer2_tc(q14, w2[:, :64], w2[:, 64:], rd, rs)

    q2 = _make_prop(2, (0, 1))(ys2, ys2, idx4)
    q2 = q2.reshape(2, _NC, _NP, 64)

    wmf2 = jnp.stack([W_mf, W_mf])
    emb, pd = _emb_tc(q2, rd, wmf2)

    recon_d = _recon_tc(emb[0, :_N])
    recon_m = _recon_tc(emb[1, :_N])

    table = jnp.concatenate([emb[0, :_N], pd[0, :_N], emb[1, :_N]], axis=0)
    pi = jnp.concatenate([tr_pos[:, 0], tr_neg[:, 0]]).astype(jnp.int32)
    mj = jnp.concatenate([tr_pos[:, 1], tr_neg[:, 1]]).astype(jnp.int32)
    gidx = jnp.concatenate([pos_cla_idx.astype(jnp.int32),
                            pi + _N, mj + 2 * _N])
    rows = _make_gather()(table, gidx)

    pred_class = _head_tc(rows[:512], W_cls0, b_cls0, W_cls, b_cls)
    preds_dm = _pairdot_tc(rows[512:512 + _G], rows[512 + _G:])[:, 0]

    return (pred_class, recon_d, recon_m, preds_dm)


# speedup vs baseline: 16.2748x; 1.0035x over previous
import functools

import jax
import jax.numpy as jnp
from jax import lax
from jax.experimental import pallas as pl
from jax.experimental.pallas import tpu as pltpu
from jax.experimental.pallas import tpu_sc as plsc

_N = 10000
_NP = 10240
_E = 320000
_NC = 2
_NS = 16
_NW = _NC * _NS
_EW = _E // _NW
_CH = 80
_NCHUNK = _EW // _CH
_RPT = _NP // _NS
_ZR = 128
_NBUF = 5

_G = 4096
_B_GATHER = 512 + 2 * _G
_BW = _B_GATHER // _NW


def _sc_mesh():
    return plsc.VectorSubcoreMesh(core_axis_name="c", subcore_axis_name="s")

def _sc_params():
    return dict(
        mesh=_sc_mesh(),
        compiler_params=pltpu.CompilerParams(use_tc_tiling_on_sc=False),
    )


def _make_hist():
    @functools.partial(
        pl.kernel,
        out_type=jax.ShapeDtypeStruct((_NC * 4 * _NP,), jnp.float32),
        scratch_types=[
            pltpu.VMEM((_NCHUNK, _CH), jnp.int32),
            pltpu.VMEM((_NCHUNK, _CH), jnp.int32),
            pltpu.VMEM((_NCHUNK, _CH), jnp.int32),
            pltpu.VMEM((_NCHUNK, _CH), jnp.int32),
            pltpu.VMEM((_CH,), jnp.float32),
            pltpu.VMEM((_RPT,), jnp.float32),
            pltpu.VMEM_SHARED((_NP,), jnp.float32),
            pltpu.VMEM_SHARED((_NP,), jnp.float32),
            pltpu.VMEM_SHARED((_NP,), jnp.float32),
            pltpu.VMEM_SHARED((_NP,), jnp.float32),
            pltpu.SemaphoreType.DMA,
        ],
        **_sc_params(),
    )
    def hist_kernel(idx_hbm, out_hbm, i0, i1, i2, i3, onesv, zv,
                    h0, h1, h2, h3, sem):
        idxs = (i0, i1, i2, i3)
        hists = (h0, h1, h2, h3)
        cid = lax.axis_index("c")
        sid = lax.axis_index("s")
        wid = cid * _NS + sid

        one16 = jnp.ones((16,), jnp.float32)
        zero16 = jnp.zeros((16,), jnp.float32)
        for k in range(_CH // 16):
            onesv[pl.ds(k * 16, 16)] = one16

        def zrow(i, carry):
            zv[pl.ds(i * 16, 16)] = zero16
            return carry
        lax.fori_loop(0, _RPT // 16, zrow, 0)
        for a in range(4):
            pltpu.sync_copy(zv, hists[a].at[pl.ds(sid * _RPT, _RPT)])
            pltpu.sync_copy(idx_hbm.at[a, wid], idxs[a])
        plsc.subcore_barrier()

        for a in range(4):
            def fire(c, carry):
                pltpu.async_copy(onesv, hists[a].at[idxs[a].at[c]], sem,
                                 add=True)
                return carry
            lax.fori_loop(0, _NCHUNK, fire, 0)

        def drain(c, carry):
            pltpu.make_async_copy(onesv, h0.at[i0.at[0]], sem).wait()
            return carry
        lax.fori_loop(0, 4 * _NCHUNK, drain, 0)

        plsc.subcore_barrier()
        for a in range(4):
            off = (cid * 4 + a) * _NP + sid * _RPT
            pltpu.sync_copy(hists[a].at[pl.ds(sid * _RPT, _RPT)],
                            out_hbm.at[pl.ds(off, _RPT)])

    return hist_kernel


def _make_prop(npass, graph):
    @functools.partial(
        pl.kernel,
        out_type=jax.ShapeDtypeStruct((npass * _NC * _NP, 64), jnp.float32),
        scratch_types=[
            pltpu.VMEM((_NCHUNK, _CH), jnp.int32),
            pltpu.VMEM((_NCHUNK, _CH), jnp.int32),
            pltpu.VMEM((_NBUF, _CH, 64), jnp.float32),
            pltpu.VMEM((_ZR, 64), jnp.float32),
            pltpu.VMEM_SHARED((_NP, 64), jnp.float32),
            [pltpu.SemaphoreType.DMA] * _NBUF,
        ],
        **_sc_params(),
    )
    def prop_kernel(ya_hbm, yb_hbm, idx_hbm, out_hbm,
                    sidx, didx, rows, zrows, acc, sems):
        cid = lax.axis_index("c")
        sid = lax.axis_index("s")
        wid = cid * _NS + sid

        zero16 = jnp.zeros((16,), jnp.float32)

        def zrow(i, carry):
            for k in range(64 // 16):
                zrows[i, pl.ds(k * 16, 16)] = zero16
            return carry
        lax.fori_loop(0, _ZR, zrow, 0)
        for r in range(_RPT // _ZR):
            pltpu.sync_copy(zrows, acc.at[pl.ds(sid * _RPT + r * _ZR, _ZR)])
        plsc.subcore_barrier()

        pltpu.sync_copy(idx_hbm.at[2 * graph, wid], sidx)
        pltpu.sync_copy(idx_hbm.at[2 * graph + 1, wid], didx)
        for p in range(npass):
            tbl = (ya_hbm if p == 0 else yb_hbm).at[graph]

            for b in range(_NBUF - 1):
                pltpu.async_copy(tbl.at[sidx.at[b]], rows.at[b], sems[b])

            def group(gi, carry):
                c0 = gi * _NBUF
                for j in range(_NBUF):
                    c = c0 + j
                    pltpu.make_async_copy(tbl.at[sidx.at[c]], rows.at[j],
                                          sems[j]).wait()
                    pltpu.sync_copy(rows.at[j], acc.at[didx.at[c]], add=True)
                    nb = (j + _NBUF - 1) % _NBUF

                    @pl.when(c + _NBUF - 1 < _NCHUNK)
                    def _():
                        pltpu.async_copy(tbl.at[sidx.at[c + _NBUF - 1]],
                                         rows.at[nb], sems[nb])
                return carry

            lax.fori_loop(0, _NCHUNK // _NBUF, group, 0)

            plsc.subcore_barrier()
            base = (p * _NC + cid) * _NP + sid * _RPT
            pltpu.sync_copy(acc.at[pl.ds(sid * _RPT, _RPT)],
                            out_hbm.at[pl.ds(base, _RPT)])
            if p + 1 < npass:
                for r in range(_RPT // _ZR):
                    pltpu.sync_copy(
                        zrows, acc.at[pl.ds(sid * _RPT + r * _ZR, _ZR)])
                plsc.subcore_barrier()

    return prop_kernel


def _make_gather():
    @functools.partial(
        pl.kernel,
        out_type=jax.ShapeDtypeStruct((_B_GATHER, 64), jnp.float32),
        scratch_types=[
            pltpu.VMEM((_BW,), jnp.int32),
            pltpu.VMEM((_BW, 64), jnp.float32),
            pltpu.SemaphoreType.DMA,
        ],
        **_sc_params(),
    )
    def gather_kernel(table_hbm, idx_hbm, out_hbm, idxv, rowsv, sem):
        cid = lax.axis_index("c")
        sid = lax.axis_index("s")
        wid = cid * _NS + sid
        base = wid * _BW
        pltpu.sync_copy(idx_hbm.at[pl.ds(base, _BW)], idxv)
        chunks = []
        for c in range(0, _BW, _CH):
            n = min(_CH, _BW - c)
            chunks.append(
                pltpu.async_copy(table_hbm.at[idxv.at[pl.ds(c, n)]],
                                 rowsv.at[pl.ds(c, n)], sem))
        for h in chunks:
            h.wait()
        pltpu.sync_copy(rowsv, out_hbm.at[pl.ds(base, _BW)])

    return gather_kernel


def _norms_tc(histp):
    def body(h_ref, o_ref):
        deg = h_ref[0] + h_ref[1]
        o_ref[...] = lax.rsqrt(deg + 1.0)
    return pl.pallas_call(
        body,
        out_shape=jax.ShapeDtypeStruct((4, _NP), jnp.float32),
    )(histp)


def _layer1_tc(x, w, rs):
    blk = 1024

    def body(x_ref, w_ref, rs_ref, oa_ref, ob_ref):
        t = jnp.dot(x_ref[0], w_ref[0],
                    preferred_element_type=jnp.float32) * rs_ref[0]
        oa_ref[0] = t[:, :64]
        ob_ref[0] = t[:, 64:]

    return pl.pallas_call(
        body,
        grid=(2, _NP // blk),
        in_specs=[
            pl.BlockSpec((1, blk, 128), lambda g, i: (g, i, 0)),
            pl.BlockSpec((1, 128, 128), lambda g, i: (g, 0, 0)),
            pl.BlockSpec((1, blk, 1), lambda g, i: (g, i, 0)),
        ],
        out_specs=[
            pl.BlockSpec((1, blk, 64), lambda g, i: (g, i, 0)),
            pl.BlockSpec((1, blk, 64), lambda g, i: (g, i, 0)),
        ],
        out_shape=[
            jax.ShapeDtypeStruct((2, _NP, 64), jnp.float32),
            jax.ShapeDtypeStruct((2, _NP, 64), jnp.float32),
        ],
    )(x, w, rs)


def _layer2_tc(q12, w2a, w2b, rdg, rsg):
    blk = 1024

    def body(qa_ref, qb_ref, wa_ref, wb_ref, rd_ref, rs_ref, o_ref):
        ha = jax.nn.relu(rd_ref[...] * (qa_ref[0, 0] + qa_ref[0, 1]))
        hb = jax.nn.relu(rd_ref[...] * (qb_ref[0, 0] + qb_ref[0, 1]))
        t = (jnp.dot(ha, wa_ref[...], preferred_element_type=jnp.float32)
             + jnp.dot(hb, wb_ref[...], preferred_element_type=jnp.float32))
        o_ref[...] = t * rs_ref[...]

    return pl.pallas_call(
        body,
        grid=(_NP // blk,),
        in_specs=[
            pl.BlockSpec((1, _NC, blk, 64), lambda i: (0, 0, i, 0)),
            pl.BlockSpec((1, _NC, blk, 64), lambda i: (1, 0, i, 0)),
            pl.BlockSpec((64, 64), lambda i: (0, 0)),
            pl.BlockSpec((64, 64), lambda i: (0, 0)),
            pl.BlockSpec((blk, 1), lambda i: (i, 0)),
            pl.BlockSpec((blk, 1), lambda i: (i, 0)),
        ],
        out_specs=pl.BlockSpec((blk, 64), lambda i: (i, 0)),
        out_shape=jax.ShapeDtypeStruct((_NP, 64), jnp.float32),
    )(q12, q12, w2a, w2b, rdg, rsg)


def _emb_tc(q2, rdg, wmf):
    blk = 1024

    def body(q_ref, rd_ref, w_ref, e_ref, p_ref):
        e = rd_ref[...] * (q_ref[0] + q_ref[1])
        e_ref[...] = e
        p_ref[...] = jnp.dot(e, w_ref[...], preferred_element_type=jnp.float32)

    return pl.pallas_call(
        body,
        grid=(_NP // blk,),
        in_specs=[
            pl.BlockSpec((_NC, blk, 64), lambda i: (0, i, 0)),
            pl.BlockSpec((blk, 1), lambda i: (i, 0)),
            pl.BlockSpec((64, 64), lambda i: (0, 0)),
        ],
        out_specs=[
            pl.BlockSpec((blk, 64), lambda i: (i, 0)),
            pl.BlockSpec((blk, 64), lambda i: (i, 0)),
        ],
        out_shape=[
            jax.ShapeDtypeStruct((_NP, 64), jnp.float32),
            jax.ShapeDtypeStruct((_NP, 64), jnp.float32),
        ],
    )(q2, rdg, wmf)


def _recon_tc(emb):
    bm = 400

    def body(a_ref, b_ref, o_ref):
        o_ref[...] = lax.dot_general(
            a_ref[...], b_ref[...], (((1,), (1,)), ((), ())),
            preferred_element_type=jnp.float32)

    return pl.pallas_call(
        body,
        grid=(_N // bm,),
        in_specs=[
            pl.BlockSpec((bm, 64), lambda i: (i, 0)),
            pl.BlockSpec((_N, 64), lambda i: (0, 0)),
        ],
        out_specs=pl.BlockSpec((bm, _N), lambda i: (i, 0)),
        out_shape=jax.ShapeDtypeStruct((_N, _N), jnp.float32),
    )(emb, emb)


def _head_tc(fea, w0, b0, w1, b1):
    def body(f_ref, w0_ref, b0_ref, w1_ref, b1_ref, o_ref):
        c1 = jax.nn.relu(
            jnp.dot(f_ref[...], w0_ref[...],
                    preferred_element_type=jnp.float32) + b0_ref[0])
        o_ref[...] = jnp.dot(
            c1, w1_ref[...], preferred_element_type=jnp.float32) + b1_ref[0]

    return pl.pallas_call(
        body,
        out_shape=jax.ShapeDtypeStruct((512, 14), jnp.float32),
    )(fea, w0, b0.reshape(1, -1), w1, b1.reshape(1, -1))


def _pairdot_tc(pr, mr):
    def body(a_ref, b_ref, o_ref):
        o_ref[...] = jnp.sum(a_ref[...] * b_ref[...], axis=1, keepdims=True)

    return pl.pallas_call(
        body,
        out_shape=jax.ShapeDtypeStruct((_G, 1), jnp.float32),
    )(pr, mr)


def kernel(features_d, features_m, W1_d, W2_d, W1_m, W2_m, W_cls0, b_cls0,
           W_cls, b_cls, W_mf, edge_index_d, edge_index_m, pos_cla_idx,
           tr_pos, tr_neg):
    ei = jnp.stack([edge_index_d, edge_index_m]).astype(jnp.int32)
    idx4 = ei.reshape(4, _NW, _NCHUNK, _CH)

    histp = _make_hist()(idx4).reshape(_NC, 4, _NP)
    norms = _norms_tc(histp)
    rs = jnp.stack([norms[0], norms[2]])[..., None]
    rd = jnp.stack([norms[1], norms[3]])[..., None]

    pad = ((0, 0), (0, _NP - _N), (0, 0))
    x = jnp.pad(jnp.stack([features_d, features_m]), pad)
    w1 = jnp.stack([W1_d, W1_m])
    w2 = jnp.stack([W2_d, W2_m])

    ya, yb = _layer1_tc(x, w1, rs)

    prop1_d = _make_prop(2, 0)
    prop1_m = _make_prop(2, 1)
    prop2_d = _make_prop(1, 0)
    prop2_m = _make_prop(1, 1)

    q1_d = prop1_d(ya, yb, idx4).reshape(2, _NC, _NP, 64)
    q1_m = prop1_m(ya, yb, idx4).reshape(2, _NC, _NP, 64)

    ys2_d = _layer2_tc(q1_d, w2[0, :64], w2[0, 64:], rd[0], rs[0])
    ys2_m = _layer2_tc(q1_m, w2[1, :64], w2[1, 64:], rd[1], rs[1])
    ys2 = jnp.stack([ys2_d, ys2_m])

    q2_d = prop2_d(ys2, ys2, idx4).reshape(_NC, _NP, 64)
    q2_m = prop2_m(ys2, ys2, idx4).reshape(_NC, _NP, 64)

    emb_d, pd_d = _emb_tc(q2_d, rd[0], W_mf)
    emb_m, pd_m = _emb_tc(q2_m, rd[1], W_mf)

    recon_d = _recon_tc(emb_d[:_N])
    recon_m = _recon_tc(emb_m[:_N])

    table = jnp.concatenate([emb_d[:_N], pd_d[:_N], emb_m[:_N]], axis=0)
    pi = jnp.concatenate([tr_pos[:, 0], tr_neg[:, 0]]).astype(jnp.int32)
    mj = jnp.concatenate([tr_pos[:, 1], tr_neg[:, 1]]).astype(jnp.int32)
    gidx = jnp.concatenate([pos_cla_idx.astype(jnp.int32),
                            pi + _N, mj + 2 * _N])
    rows = _make_gather()(table, gidx)

    pred_class = _head_tc(rows[:512], W_cls0, b_cls0, W_cls, b_cls)
    preds_dm = _pairdot_tc(rows[512:512 + _G], rows[512 + _G:])[:, 0]

    return (pred_class, recon_d, recon_m, preds_dm)
